# Initial kernel scaffold; baseline (speedup 1.0000x reference)
import functools

import jax
import jax.numpy as jnp
from jax import lax
from jax.experimental import pallas as pl
from jax.experimental.pallas import tpu as pltpu
from jax.experimental.pallas import tpu_sc as plsc

N_ROWS = 320000
D = 128
N_SEG = 10000
SEG_PAD = 10240
ROWS_PER_TILE_OUT = SEG_PAD // 16
CH = 128
N_CHUNKS = N_ROWS // CH
NC = 2
NS = 16
NW = NC * NS


def _sc_body(x_hbm, idx_hbm, out_s, out_c, idx_v, rows_v, ones_v, zcnt_v,
             sums_sh, cnts_sh):
    c = lax.axis_index("c")
    s = lax.axis_index("s")
    wid = c * NS + s

    zeros16 = jnp.zeros((16,), jnp.float32)
    ones16 = jnp.ones((16,), jnp.float32)

    def fill_rows(i, _):
        for j in range(D // 16):
            rows_v[i, pl.ds(j * 16, 16)] = zeros16
        ones_v[i, :] = ones16
        return 0
    lax.fori_loop(0, CH, fill_rows, 0)

    def fill_zcnt(i, _):
        zcnt_v[i, :] = zeros16
        return 0
    lax.fori_loop(0, ROWS_PER_TILE_OUT, fill_zcnt, 0)

    for k in range(ROWS_PER_TILE_OUT // CH):
        pltpu.sync_copy(rows_v, sums_sh.at[pl.ds(s * ROWS_PER_TILE_OUT + k * CH, CH)])
    pltpu.sync_copy(zcnt_v, cnts_sh.at[pl.ds(s * ROWS_PER_TILE_OUT, ROWS_PER_TILE_OUT)])

    plsc.subcore_barrier()

    start = (wid * N_CHUNKS) // NW
    end = ((wid + 1) * N_CHUNKS) // NW

    def chunk_body(ci, _):
        base = ci * CH
        pltpu.sync_copy(idx_hbm.at[pl.ds(base, CH)], idx_v)
        pltpu.sync_copy(x_hbm.at[pl.ds(base, CH)], rows_v)
        pltpu.sync_copy(rows_v, sums_sh.at[idx_v], add=True)
        pltpu.sync_copy(ones_v, cnts_sh.at[idx_v], add=True)
        return 0
    lax.fori_loop(start, end, chunk_body, 0)

    plsc.subcore_barrier()

    row0 = c * SEG_PAD + s * ROWS_PER_TILE_OUT
    pltpu.sync_copy(sums_sh.at[pl.ds(s * ROWS_PER_TILE_OUT, ROWS_PER_TILE_OUT)],
                    out_s.at[pl.ds(row0, ROWS_PER_TILE_OUT)])
    pltpu.sync_copy(cnts_sh.at[pl.ds(s * ROWS_PER_TILE_OUT, ROWS_PER_TILE_OUT)],
                    out_c.at[pl.ds(row0, ROWS_PER_TILE_OUT)])


_sc_scatter = pl.kernel(
    _sc_body,
    out_type=(
        jax.ShapeDtypeStruct((NC * SEG_PAD, D), jnp.float32),
        jax.ShapeDtypeStruct((NC * SEG_PAD, 16), jnp.float32),
    ),
    mesh=plsc.VectorSubcoreMesh(core_axis_name="c", subcore_axis_name="s"),
    scratch_types=[
        pltpu.VMEM((CH,), jnp.int32),
        pltpu.VMEM((CH, D), jnp.float32),
        pltpu.VMEM((CH, 16), jnp.float32),
        pltpu.VMEM((ROWS_PER_TILE_OUT, 16), jnp.float32),
        pltpu.VMEM_SHARED((SEG_PAD, D), jnp.float32),
        pltpu.VMEM_SHARED((SEG_PAD, 16), jnp.float32),
    ],
)

_BLK = 512
_NBLK = SEG_PAD // _BLK


def _combine_body(sa, sb, ca, cb, out):
    cnt = ca[:, 0:1] + cb[:, 0:1]
    out[...] = (sa[...] + sb[...]) / jnp.maximum(cnt, 1.0)


def _combine(sums, cnts):
    return pl.pallas_call(
        _combine_body,
        grid=(_NBLK,),
        in_specs=[
            pl.BlockSpec((_BLK, D), lambda i: (i, 0)),
            pl.BlockSpec((_BLK, D), lambda i: (i + _NBLK, 0)),
            pl.BlockSpec((_BLK, 16), lambda i: (i, 0)),
            pl.BlockSpec((_BLK, 16), lambda i: (i + _NBLK, 0)),
        ],
        out_specs=pl.BlockSpec((_BLK, D), lambda i: (i, 0)),
        out_shape=jax.ShapeDtypeStruct((SEG_PAD, D), jnp.float32),
    )(sums, sums, cnts, cnts)


@jax.jit
def kernel(x, molecule_idx):
    idx32 = molecule_idx.astype(jnp.int32)
    sums, cnts = _sc_scatter(x, idx32)
    out = _combine(sums, cnts)
    return out[:N_SEG]


# speedup vs baseline: 3.9254x; 3.9254x over previous
import jax
import jax.numpy as jnp
from jax import lax
from jax.experimental import pallas as pl
from jax.experimental.pallas import tpu as pltpu
from jax.experimental.pallas import tpu_sc as plsc

N_ROWS = 320000
D = 128
DH = D // 2
N_SEG = 10000
SEG_PAD = 10240
RPT = SEG_PAD // 16
CH = 128
N_CHUNKS = N_ROWS // CH
NC = 2
NS = 16


def _sc_body(x_hbm, idx_hbm, out_s, out_c, idx_v, rows_v, ones_v, zcnt_v,
             sums_sh, cnts_sh):
    c = lax.axis_index("c")
    s = lax.axis_index("s")

    zeros16 = jnp.zeros((16,), jnp.float32)
    ones16 = jnp.ones((16,), jnp.float32)

    def fill_rows(i, _):
        for j in range(DH // 16):
            rows_v[i, pl.ds(j * 16, 16)] = zeros16
        ones_v[i, :] = ones16
        return 0
    lax.fori_loop(0, CH, fill_rows, 0)

    def fill_zcnt(i, _):
        zcnt_v[i, :] = zeros16
        return 0
    lax.fori_loop(0, RPT, fill_zcnt, 0)

    for k in range(RPT // CH):
        pltpu.sync_copy(rows_v, sums_sh.at[pl.ds(s * RPT + k * CH, CH)])
    pltpu.sync_copy(zcnt_v, cnts_sh.at[pl.ds(s * RPT, RPT)])

    plsc.subcore_barrier()

    start = (s * N_CHUNKS) // NS
    end = ((s + 1) * N_CHUNKS) // NS

    def chunk_body(ci, _):
        base = ci * CH
        pltpu.sync_copy(idx_hbm.at[pl.ds(base, CH)], idx_v)

        @pl.when(c == 0)
        def _():
            pltpu.sync_copy(x_hbm.at[pl.ds(base, CH), pl.ds(0, DH)], rows_v)
            pltpu.sync_copy(ones_v, cnts_sh.at[idx_v], add=True)

        @pl.when(c == 1)
        def _():
            pltpu.sync_copy(x_hbm.at[pl.ds(base, CH), pl.ds(DH, DH)], rows_v)

        pltpu.sync_copy(rows_v, sums_sh.at[idx_v], add=True)
        return 0
    lax.fori_loop(start, end, chunk_body, 0)

    plsc.subcore_barrier()

    pltpu.sync_copy(sums_sh.at[pl.ds(s * RPT, RPT)],
                    out_s.at[pl.ds(c * SEG_PAD + s * RPT, RPT)])

    @pl.when(c == 0)
    def _():
        pltpu.sync_copy(cnts_sh.at[pl.ds(s * RPT, RPT)],
                        out_c.at[pl.ds(s * RPT, RPT)])


_sc_scatter = pl.kernel(
    _sc_body,
    out_type=(
        jax.ShapeDtypeStruct((NC * SEG_PAD, DH), jnp.float32),
        jax.ShapeDtypeStruct((SEG_PAD, 16), jnp.float32),
    ),
    mesh=plsc.VectorSubcoreMesh(core_axis_name="c", subcore_axis_name="s"),
    compiler_params=pltpu.CompilerParams(use_tc_tiling_on_sc=False),
    scratch_types=[
        pltpu.VMEM((CH,), jnp.int32),
        pltpu.VMEM((CH, DH), jnp.float32),
        pltpu.VMEM((CH, 16), jnp.float32),
        pltpu.VMEM((RPT, 16), jnp.float32),
        pltpu.VMEM_SHARED((SEG_PAD, DH), jnp.float32),
        pltpu.VMEM_SHARED((SEG_PAD, 16), jnp.float32),
    ],
)

_BLK = 512
_NBLK = SEG_PAD // _BLK


def _combine_body(sa, sb, ca, out):
    inv = 1.0 / jnp.maximum(ca[:, 0:1], 1.0)
    out[...] = jnp.concatenate([sa[...], sb[...]], axis=1) * inv


def _combine(sums, cnts):
    return pl.pallas_call(
        _combine_body,
        grid=(_NBLK,),
        in_specs=[
            pl.BlockSpec((_BLK, DH), lambda i: (i, 0)),
            pl.BlockSpec((_BLK, DH), lambda i: (i + _NBLK, 0)),
            pl.BlockSpec((_BLK, 16), lambda i: (i, 0)),
        ],
        out_specs=pl.BlockSpec((_BLK, D), lambda i: (i, 0)),
        out_shape=jax.ShapeDtypeStruct((SEG_PAD, D), jnp.float32),
    )(sums, sums, cnts)


@jax.jit
def kernel(x, molecule_idx):
    idx32 = molecule_idx.astype(jnp.int32)
    sums, cnts = _sc_scatter(x, idx32)
    out = _combine(sums, cnts)
    return out[:N_SEG]
